# baseline (speedup 1.0000x reference)
import functools
import jax
import jax.numpy as jnp
from jax import lax
from jax.experimental import pallas as pl
from jax.experimental.pallas import tpu as pltpu
from jax.experimental.pallas import tpu_sc as plsc

N = 10000
NP = 10240
F = 256
HALF = 128
NH = 8
G = 64
NCLS = 40
EFULL = 170000
T = 512
K1 = 11
EP = 32 * T * K1
K2 = EP // (16 * T)
ROWS_PT = NP // 16
DUMMY = N

_R = 512
_GRID = NP // _R


def _tc_first_body(x_ref, w_ref, ms_ref, md_ref, hlo_ref, hhi_ref, as_ref, ad_ref):
    h = jnp.dot(x_ref[...], w_ref[...], preferred_element_type=jnp.float32)
    hlo_ref[...] = h[:, :HALF]
    hhi_ref[...] = h[:, HALF:]
    as_ref[...] = jnp.dot(h, ms_ref[...], preferred_element_type=jnp.float32)
    ad_ref[...] = jnp.dot(h, md_ref[...], preferred_element_type=jnp.float32)


def _tc_mid_body(plo_ref, phi_ref, b_ref, w_ref, ms_ref, md_ref,
                 hlo_ref, hhi_ref, as_ref, ad_ref):
    b = b_ref[...]
    alo = plo_ref[...] + b[:, :HALF]
    ahi = phi_ref[...] + b[:, HALF:]
    alo = jnp.where(alo > 0, alo, jnp.exp(alo) - 1.0)
    ahi = jnp.where(ahi > 0, ahi, jnp.exp(ahi) - 1.0)
    h = (jnp.dot(alo, w_ref[:HALF, :], preferred_element_type=jnp.float32)
         + jnp.dot(ahi, w_ref[HALF:, :], preferred_element_type=jnp.float32))
    hlo_ref[...] = h[:, :HALF]
    hhi_ref[...] = h[:, HALF:]
    as_ref[...] = jnp.dot(h, ms_ref[...], preferred_element_type=jnp.float32)
    ad_ref[...] = jnp.dot(h, md_ref[...], preferred_element_type=jnp.float32)


_TC_OUT_SHAPE = [jax.ShapeDtypeStruct((NP, HALF), jnp.float32),
                 jax.ShapeDtypeStruct((NP, HALF), jnp.float32),
                 jax.ShapeDtypeStruct((NP, 16), jnp.float32),
                 jax.ShapeDtypeStruct((NP, 16), jnp.float32)]
_TC_OUT_SPECS = [pl.BlockSpec((_R, HALF), lambda i: (i, 0)),
                 pl.BlockSpec((_R, HALF), lambda i: (i, 0)),
                 pl.BlockSpec((_R, 16), lambda i: (i, 0)),
                 pl.BlockSpec((_R, 16), lambda i: (i, 0))]


def _tc_first(xp, W, Ms, Md):
    return pl.pallas_call(
        _tc_first_body,
        grid=(_GRID,),
        in_specs=[pl.BlockSpec((_R, F), lambda i: (i, 0)),
                  pl.BlockSpec((F, F), lambda i: (0, 0)),
                  pl.BlockSpec((F, 16), lambda i: (0, 0)),
                  pl.BlockSpec((F, 16), lambda i: (0, 0))],
        out_specs=_TC_OUT_SPECS,
        out_shape=_TC_OUT_SHAPE,
    )(xp, W, Ms, Md)


def _tc_mid(plo, phi, b, W, Ms, Md):
    return pl.pallas_call(
        _tc_mid_body,
        grid=(_GRID,),
        in_specs=[pl.BlockSpec((_R, HALF), lambda i: (i, 0)),
                  pl.BlockSpec((_R, HALF), lambda i: (i, 0)),
                  pl.BlockSpec((1, F), lambda i: (0, 0)),
                  pl.BlockSpec((F, F), lambda i: (0, 0)),
                  pl.BlockSpec((F, 16), lambda i: (0, 0)),
                  pl.BlockSpec((F, 16), lambda i: (0, 0))],
        out_specs=_TC_OUT_SPECS,
        out_shape=_TC_OUT_SHAPE,
    )(plo, phi, b, W, Ms, Md)


def _pool_body(plo_ref, phi_ref, b_ref, batch_ref, fcw_ref, fcb_ref, out_ref,
               sum_ref, cnt_ref):
    i = pl.program_id(0)

    @pl.when(i == 0)
    def _():
        sum_ref[...] = jnp.zeros_like(sum_ref)
        cnt_ref[...] = jnp.zeros_like(cnt_ref)

    b = b_ref[...]
    y = jnp.concatenate([plo_ref[...] + b[:, :HALF], phi_ref[...] + b[:, HALF:]],
                        axis=1)
    bb = batch_ref[0]
    gi = lax.broadcasted_iota(jnp.int32, (G, _R), 0)
    oh = (gi == bb).astype(jnp.float32)
    sum_ref[...] += jnp.dot(oh, y, preferred_element_type=jnp.float32)
    cnt_ref[...] += jnp.broadcast_to(jnp.sum(oh, axis=1, keepdims=True), (G, HALF))

    @pl.when(i == _GRID - 1)
    def _():
        cnt = jnp.maximum(cnt_ref[...], 1.0)
        pooled = sum_ref[...] / jnp.concatenate([cnt, cnt], axis=1)
        out_ref[...] = (jnp.dot(pooled, fcw_ref[...],
                                preferred_element_type=jnp.float32) + fcb_ref[...])


def _tc_pool(plo, phi, b, batch3d, fcW, fcb):
    return pl.pallas_call(
        _pool_body,
        grid=(_GRID,),
        in_specs=[pl.BlockSpec((_R, HALF), lambda i: (i, 0)),
                  pl.BlockSpec((_R, HALF), lambda i: (i, 0)),
                  pl.BlockSpec((1, F), lambda i: (0, 0)),
                  pl.BlockSpec((1, 1, _R), lambda i: (i, 0, 0)),
                  pl.BlockSpec((F, NCLS), lambda i: (0, 0)),
                  pl.BlockSpec((1, NCLS), lambda i: (0, 0))],
        out_specs=pl.BlockSpec((G, NCLS), lambda i: (0, 0)),
        out_shape=jax.ShapeDtypeStruct((G, NCLS), jnp.float32),
        scratch_shapes=[pltpu.VMEM((G, F), jnp.float32),
                        pltpu.VMEM((G, HALF), jnp.float32)],
    )(plo, phi, b, batch3d, fcW, fcb)


_MESH = plsc.VectorSubcoreMesh(core_axis_name="c", subcore_axis_name="s")


def _denr_body(d0_ref, d1_ref, out_ref):
    out_ref[...] = 1.0 / (d0_ref[...] + d1_ref[...] + 1e-16)


def _tc_denr(d0, d1):
    return pl.pallas_call(
        _denr_body,
        grid=(_GRID,),
        in_specs=[pl.BlockSpec((_R, 16), lambda i: (i, 0)),
                  pl.BlockSpec((_R, 16), lambda i: (i, 0))],
        out_specs=pl.BlockSpec((_R, 16), lambda i: (i, 0)),
        out_shape=jax.ShapeDtypeStruct((NP, 16), jnp.float32),
    )(d0, d1)


@functools.partial(
    pl.kernel,
    out_type=[jax.ShapeDtypeStruct((EP, 16), jnp.float32),
              jax.ShapeDtypeStruct((2, NP, 16), jnp.float32)],
    mesh=_MESH,
    scratch_types=[pltpu.VMEM((4, T // 4), jnp.int32),
                   pltpu.VMEM((4, T // 4), jnp.int32),
                   pltpu.VMEM((T, 16), jnp.float32),
                   pltpu.VMEM((T, 16), jnp.float32),
                   pltpu.VMEM_SHARED((NP, 16), jnp.float32)],
    compiler_params=pltpu.CompilerParams(use_tc_tiling_on_sc=False, needs_layout_passes=False),
)
def _sc_pass1(src_hbm, dst_hbm, as_hbm, ad_hbm, zden_hbm, t_hbm, den_hbm,
              src_v, dst_v, srow, drow, den_sh):
    c = lax.axis_index("c")
    s = lax.axis_index("s")
    wid = s * 2 + c
    pltpu.sync_copy(zden_hbm.at[pl.ds(s * ROWS_PT, ROWS_PT), :],
                    den_sh.at[pl.ds(s * ROWS_PT, ROWS_PT), :])
    plsc.subcore_barrier()
    wbase = wid * (T * K1)

    def chunk(k, carry):
        base = wbase + k * T
        brow = base // 128
        pltpu.sync_copy(src_hbm.at[pl.ds(brow, T // 128), :], src_v)
        pltpu.sync_copy(dst_hbm.at[pl.ds(brow, T // 128), :], dst_v)
        for q in range(T // 128):
            pltpu.sync_copy(as_hbm.at[src_v.at[q]],
                            srow.at[pl.ds(q * 128, 128), :])
            pltpu.sync_copy(ad_hbm.at[dst_v.at[q]],
                            drow.at[pl.ds(q * 128, 128), :])

        def _row(i, cr):
            v = srow[i, :] + drow[i, :]
            srow[i, :] = jnp.exp(jnp.maximum(v, 0.2 * v))
            return cr

        lax.fori_loop(0, T, _row, 0)

        pltpu.sync_copy(srow, t_hbm.at[pl.ds(base, T), :])
        for q in range(T // 128):
            pltpu.sync_copy(srow.at[pl.ds(q * 128, 128), :],
                            den_sh.at[dst_v.at[q]], add=True)
        return carry

    lax.fori_loop(0, K1, chunk, 0)
    plsc.subcore_barrier()
    pltpu.sync_copy(den_sh.at[pl.ds(s * ROWS_PT, ROWS_PT), :],
                    den_hbm.at[c, pl.ds(s * ROWS_PT, ROWS_PT), :])


@functools.partial(
    pl.kernel,
    out_type=jax.ShapeDtypeStruct((2, NP, HALF), jnp.float32),
    mesh=_MESH,
    scratch_types=[pltpu.VMEM((4, T // 4), jnp.int32),
                   pltpu.VMEM((4, T // 4), jnp.int32),
                   pltpu.VMEM((T, 16), jnp.float32),
                   pltpu.VMEM((T, 16), jnp.float32),
                   pltpu.VMEM((128, HALF), jnp.float32),
                   pltpu.VMEM_SHARED((NP, HALF), jnp.float32)],
    compiler_params=pltpu.CompilerParams(use_tc_tiling_on_sc=False, needs_layout_passes=False),
)
def _sc_pass2(src_hbm, dst_hbm, t_hbm, denr_hbm, hlo_hbm, hhi_hbm,
              zacc_hbm, out_hbm,
              src_v, dst_v, trow, dr, hbuf, acc_sh):
    c = lax.axis_index("c")
    s = lax.axis_index("s")
    pltpu.sync_copy(zacc_hbm.at[pl.ds(s * ROWS_PT, ROWS_PT), :],
                    acc_sh.at[pl.ds(s * ROWS_PT, ROWS_PT), :])
    plsc.subcore_barrier()
    tbase = s * (T * K2)
    hb = c * 4
    cols = [jnp.full((16,), hb + jj, jnp.int32) for jj in range(4)]

    def chunk(k, carry):
        base = tbase + k * T
        brow = base // 128
        pltpu.sync_copy(src_hbm.at[pl.ds(brow, T // 128), :], src_v)
        pltpu.sync_copy(dst_hbm.at[pl.ds(brow, T // 128), :], dst_v)
        pltpu.sync_copy(t_hbm.at[pl.ds(base, T), :], trow)
        for q in range(T // 128):
            pltpu.sync_copy(denr_hbm.at[dst_v.at[q]],
                            dr.at[pl.ds(q * 128, 128), :])

        def _coef_row(i, cr):
            trow[i, :] = trow[i, :] * dr[i, :]
            return cr

        lax.fori_loop(0, T, _coef_row, 0)

        for q in range(T // 128):
            @pl.when(c == 0)
            def _():
                pltpu.sync_copy(hlo_hbm.at[src_v.at[q]], hbuf)

            @pl.when(c != 0)
            def _():
                pltpu.sync_copy(hhi_hbm.at[src_v.at[q]], hbuf)

            def _scale_row(i, cr):
                rowi = jnp.full((16,), q * 128 + i, jnp.int32)
                for jj in range(4):
                    ce = plsc.load_gather(trow, [rowi, cols[jj]])
                    hbuf[i, pl.ds(jj * 32, 16)] = hbuf[i, pl.ds(jj * 32, 16)] * ce
                    hbuf[i, pl.ds(jj * 32 + 16, 16)] = (
                        hbuf[i, pl.ds(jj * 32 + 16, 16)] * ce)
                return cr

            lax.fori_loop(0, 128, _scale_row, 0)

            pltpu.sync_copy(hbuf, acc_sh.at[dst_v.at[q]], add=True)
        return carry

    lax.fori_loop(0, K2, chunk, 0)
    plsc.subcore_barrier()
    pltpu.sync_copy(acc_sh.at[pl.ds(s * ROWS_PT, ROWS_PT), :],
                    out_hbm.at[c, pl.ds(s * ROWS_PT, ROWS_PT), :])


def kernel(x, edge_index, batch, W1, as1, ad1, b1, W2, as2, ad2, b2,
           W3, as3, ad3, b3, fcW, fcb):
    f32 = jnp.float32
    loops = jnp.arange(N, dtype=jnp.int32)
    src = jnp.concatenate([edge_index[0].astype(jnp.int32), loops,
                           jnp.zeros((EP - EFULL,), jnp.int32)]).reshape(EP // 128, 128)
    dst = jnp.concatenate([edge_index[1].astype(jnp.int32), loops,
                           jnp.full((EP - EFULL,), DUMMY, jnp.int32)]).reshape(EP // 128, 128)
    xp = jnp.pad(x, ((0, NP - N), (0, 0)))
    zden = jnp.zeros((NP, 16), f32)
    zacc = jnp.zeros((NP, HALF), f32)
    eye8 = jnp.eye(NH, dtype=f32)

    def amat(a):
        m = (a[:, :, None] * eye8[:, None, :]).reshape(F, NH)
        return jnp.concatenate([m, m], axis=1)

    batchp = jnp.concatenate([batch.astype(jnp.int32),
                              jnp.full((NP - N,), G, jnp.int32)])
    batch3d = batchp.reshape(_GRID, 1, _R)

    hlo, hhi, As, Ad = _tc_first(xp, W1, amat(as1), amat(ad1))
    for (W, a_s, a_d, b_) in ((W2, as2, ad2, b1), (W3, as3, ad3, b2)):
        t_buf, den = _sc_pass1(src, dst, As, Ad, zden)
        out = _sc_pass2(src, dst, t_buf, _tc_denr(den[0], den[1]), hlo, hhi, zacc)
        hlo, hhi, As, Ad = _tc_mid(out[0], out[1], b_.reshape(1, F), W,
                                   amat(a_s), amat(a_d))
    t_buf, den = _sc_pass1(src, dst, As, Ad, zden)
    out = _sc_pass2(src, dst, t_buf, _tc_denr(den[0], den[1]), hlo, hhi, zacc)
    return _tc_pool(out[0], out[1], b3.reshape(1, F), batch3d, fcW,
                    fcb.reshape(1, NCLS))


# speedup vs baseline: 1.1332x; 1.1332x over previous
import functools
import jax
import jax.numpy as jnp
from jax import lax
from jax.experimental import pallas as pl
from jax.experimental.pallas import tpu as pltpu
from jax.experimental.pallas import tpu_sc as plsc

N = 10000
NP = 10240
F = 256
HALF = 128
NH = 8
G = 64
NCLS = 40
EFULL = 170000
T = 512
K1 = 11
EP = 32 * T * K1
K2 = EP // (16 * T)
ROWS_PT = NP // 16
DUMMY = N

_R = 512
_GRID = NP // _R


def _tc_first_body(x_ref, w_ref, ms_ref, md_ref, hlo_ref, hhi_ref, as_ref, ad_ref):
    h = jnp.dot(x_ref[...], w_ref[...], preferred_element_type=jnp.float32)
    hlo_ref[...] = h[:, :HALF]
    hhi_ref[...] = h[:, HALF:]
    as_ref[...] = jnp.dot(h, ms_ref[...], preferred_element_type=jnp.float32)
    ad_ref[...] = jnp.dot(h, md_ref[...], preferred_element_type=jnp.float32)


def _tc_mid_body(plo_ref, phi_ref, b_ref, w_ref, ms_ref, md_ref,
                 hlo_ref, hhi_ref, as_ref, ad_ref):
    b = b_ref[...]
    alo = plo_ref[...] + b[:, :HALF]
    ahi = phi_ref[...] + b[:, HALF:]
    alo = jnp.where(alo > 0, alo, jnp.exp(alo) - 1.0)
    ahi = jnp.where(ahi > 0, ahi, jnp.exp(ahi) - 1.0)
    h = (jnp.dot(alo, w_ref[:HALF, :], preferred_element_type=jnp.float32)
         + jnp.dot(ahi, w_ref[HALF:, :], preferred_element_type=jnp.float32))
    hlo_ref[...] = h[:, :HALF]
    hhi_ref[...] = h[:, HALF:]
    as_ref[...] = jnp.dot(h, ms_ref[...], preferred_element_type=jnp.float32)
    ad_ref[...] = jnp.dot(h, md_ref[...], preferred_element_type=jnp.float32)


_TC_OUT_SHAPE = [jax.ShapeDtypeStruct((NP, HALF), jnp.float32),
                 jax.ShapeDtypeStruct((NP, HALF), jnp.float32),
                 jax.ShapeDtypeStruct((NP, 16), jnp.float32),
                 jax.ShapeDtypeStruct((NP, 16), jnp.float32)]
_TC_OUT_SPECS = [pl.BlockSpec((_R, HALF), lambda i: (i, 0)),
                 pl.BlockSpec((_R, HALF), lambda i: (i, 0)),
                 pl.BlockSpec((_R, 16), lambda i: (i, 0)),
                 pl.BlockSpec((_R, 16), lambda i: (i, 0))]


def _tc_first(xp, W, Ms, Md):
    return pl.pallas_call(
        _tc_first_body,
        grid=(_GRID,),
        in_specs=[pl.BlockSpec((_R, F), lambda i: (i, 0)),
                  pl.BlockSpec((F, F), lambda i: (0, 0)),
                  pl.BlockSpec((F, 16), lambda i: (0, 0)),
                  pl.BlockSpec((F, 16), lambda i: (0, 0))],
        out_specs=_TC_OUT_SPECS,
        out_shape=_TC_OUT_SHAPE,
    )(xp, W, Ms, Md)


def _tc_mid(plo, phi, b, W, Ms, Md):
    return pl.pallas_call(
        _tc_mid_body,
        grid=(_GRID,),
        in_specs=[pl.BlockSpec((_R, HALF), lambda i: (i, 0)),
                  pl.BlockSpec((_R, HALF), lambda i: (i, 0)),
                  pl.BlockSpec((1, F), lambda i: (0, 0)),
                  pl.BlockSpec((F, F), lambda i: (0, 0)),
                  pl.BlockSpec((F, 16), lambda i: (0, 0)),
                  pl.BlockSpec((F, 16), lambda i: (0, 0))],
        out_specs=_TC_OUT_SPECS,
        out_shape=_TC_OUT_SHAPE,
    )(plo, phi, b, W, Ms, Md)


def _pool_body(plo_ref, phi_ref, b_ref, batch_ref, fcw_ref, fcb_ref, out_ref,
               sum_ref, cnt_ref):
    i = pl.program_id(0)

    @pl.when(i == 0)
    def _():
        sum_ref[...] = jnp.zeros_like(sum_ref)
        cnt_ref[...] = jnp.zeros_like(cnt_ref)

    b = b_ref[...]
    y = jnp.concatenate([plo_ref[...] + b[:, :HALF], phi_ref[...] + b[:, HALF:]],
                        axis=1)
    bb = batch_ref[0]
    gi = lax.broadcasted_iota(jnp.int32, (G, _R), 0)
    oh = (gi == bb).astype(jnp.float32)
    sum_ref[...] += jnp.dot(oh, y, preferred_element_type=jnp.float32)
    cnt_ref[...] += jnp.broadcast_to(jnp.sum(oh, axis=1, keepdims=True), (G, HALF))

    @pl.when(i == _GRID - 1)
    def _():
        cnt = jnp.maximum(cnt_ref[...], 1.0)
        pooled = sum_ref[...] / jnp.concatenate([cnt, cnt], axis=1)
        out_ref[...] = (jnp.dot(pooled, fcw_ref[...],
                                preferred_element_type=jnp.float32) + fcb_ref[...])


def _tc_pool(plo, phi, b, batch3d, fcW, fcb):
    return pl.pallas_call(
        _pool_body,
        grid=(_GRID,),
        in_specs=[pl.BlockSpec((_R, HALF), lambda i: (i, 0)),
                  pl.BlockSpec((_R, HALF), lambda i: (i, 0)),
                  pl.BlockSpec((1, F), lambda i: (0, 0)),
                  pl.BlockSpec((1, 1, _R), lambda i: (i, 0, 0)),
                  pl.BlockSpec((F, NCLS), lambda i: (0, 0)),
                  pl.BlockSpec((1, NCLS), lambda i: (0, 0))],
        out_specs=pl.BlockSpec((G, NCLS), lambda i: (0, 0)),
        out_shape=jax.ShapeDtypeStruct((G, NCLS), jnp.float32),
        scratch_shapes=[pltpu.VMEM((G, F), jnp.float32),
                        pltpu.VMEM((G, HALF), jnp.float32)],
    )(plo, phi, b, batch3d, fcW, fcb)


_MESH = plsc.VectorSubcoreMesh(core_axis_name="c", subcore_axis_name="s")


def _denr_body(d0_ref, d1_ref, out_ref):
    out_ref[...] = 1.0 / (d0_ref[...] + d1_ref[...] + 1e-16)


def _tc_denr(d0, d1):
    return pl.pallas_call(
        _denr_body,
        grid=(_GRID,),
        in_specs=[pl.BlockSpec((_R, 16), lambda i: (i, 0)),
                  pl.BlockSpec((_R, 16), lambda i: (i, 0))],
        out_specs=pl.BlockSpec((_R, 16), lambda i: (i, 0)),
        out_shape=jax.ShapeDtypeStruct((NP, 16), jnp.float32),
    )(d0, d1)


@functools.partial(
    pl.kernel,
    out_type=[jax.ShapeDtypeStruct((EP, 16), jnp.float32),
              jax.ShapeDtypeStruct((2, NP, 16), jnp.float32)],
    mesh=_MESH,
    scratch_types=[pltpu.VMEM((4, T // 4), jnp.int32),
                   pltpu.VMEM((4, T // 4), jnp.int32),
                   pltpu.VMEM((T, 16), jnp.float32),
                   pltpu.VMEM((T, 16), jnp.float32),
                   pltpu.VMEM_SHARED((NP, 16), jnp.float32)],
    compiler_params=pltpu.CompilerParams(use_tc_tiling_on_sc=False, needs_layout_passes=False),
)
def _sc_pass1(src_hbm, dst_hbm, as_hbm, ad_hbm, zden_hbm, t_hbm, den_hbm,
              src_v, dst_v, srow, drow, den_sh):
    c = lax.axis_index("c")
    s = lax.axis_index("s")
    wid = s * 2 + c
    pltpu.sync_copy(zden_hbm.at[pl.ds(s * ROWS_PT, ROWS_PT), :],
                    den_sh.at[pl.ds(s * ROWS_PT, ROWS_PT), :])
    plsc.subcore_barrier()
    wbase = wid * (T * K1)

    def chunk(k, carry):
        base = wbase + k * T
        brow = base // 128
        pltpu.sync_copy(src_hbm.at[pl.ds(brow, T // 128), :], src_v)
        pltpu.sync_copy(dst_hbm.at[pl.ds(brow, T // 128), :], dst_v)
        for q in range(T // 128):
            pltpu.sync_copy(as_hbm.at[src_v.at[q]],
                            srow.at[pl.ds(q * 128, 128), :])
            pltpu.sync_copy(ad_hbm.at[dst_v.at[q]],
                            drow.at[pl.ds(q * 128, 128), :])

        def _row(i, cr):
            v = srow[i, :] + drow[i, :]
            srow[i, :] = jnp.exp(jnp.maximum(v, 0.2 * v))
            return cr

        lax.fori_loop(0, T, _row, 0)

        pltpu.sync_copy(srow, t_hbm.at[pl.ds(base, T), :])
        for q in range(T // 128):
            pltpu.sync_copy(srow.at[pl.ds(q * 128, 128), :],
                            den_sh.at[dst_v.at[q]], add=True)
        return carry

    lax.fori_loop(0, K1, chunk, 0)
    plsc.subcore_barrier()
    pltpu.sync_copy(den_sh.at[pl.ds(s * ROWS_PT, ROWS_PT), :],
                    den_hbm.at[c, pl.ds(s * ROWS_PT, ROWS_PT), :])


@functools.partial(
    pl.kernel,
    out_type=jax.ShapeDtypeStruct((2, NP, HALF), jnp.float32),
    mesh=_MESH,
    scratch_types=[pltpu.VMEM((4, T // 4), jnp.int32),
                   pltpu.VMEM((4, T // 4), jnp.int32),
                   pltpu.VMEM((T, 16), jnp.float32),
                   pltpu.VMEM((T, 16), jnp.float32),
                   pltpu.VMEM((128, HALF), jnp.float32),
                   pltpu.VMEM_SHARED((NP, HALF), jnp.float32)],
    compiler_params=pltpu.CompilerParams(use_tc_tiling_on_sc=False, needs_layout_passes=False),
)
def _sc_pass2(src_hbm, dst_hbm, t_hbm, denr_hbm, hlo_hbm, hhi_hbm,
              zacc_hbm, out_hbm,
              src_v, dst_v, trow, dr, hbuf, acc_sh):
    c = lax.axis_index("c")
    s = lax.axis_index("s")
    pltpu.sync_copy(zacc_hbm.at[pl.ds(s * ROWS_PT, ROWS_PT), :],
                    acc_sh.at[pl.ds(s * ROWS_PT, ROWS_PT), :])
    plsc.subcore_barrier()
    tbase = s * (T * K2)
    hb = c * 4
    cols = [jnp.full((16,), hb + jj, jnp.int32) for jj in range(4)]

    def chunk(k, carry):
        base = tbase + k * T
        brow = base // 128
        pltpu.sync_copy(src_hbm.at[pl.ds(brow, T // 128), :], src_v)
        pltpu.sync_copy(dst_hbm.at[pl.ds(brow, T // 128), :], dst_v)
        pltpu.sync_copy(t_hbm.at[pl.ds(base, T), :], trow)
        for q in range(T // 128):
            pltpu.sync_copy(denr_hbm.at[dst_v.at[q]],
                            dr.at[pl.ds(q * 128, 128), :])

        def _coef_row(i, cr):
            trow[i, :] = trow[i, :] * dr[i, :]
            return cr

        lax.fori_loop(0, T, _coef_row, 0)

        for q in range(T // 128):
            @pl.when(c == 0)
            def _():
                pltpu.sync_copy(hlo_hbm.at[src_v.at[q]], hbuf)

            @pl.when(c != 0)
            def _():
                pltpu.sync_copy(hhi_hbm.at[src_v.at[q]], hbuf)

            @plsc.parallel_loop(0, 128)
            def _scale_row(i):
                rowi = jnp.full((16,), q * 128 + i, jnp.int32)
                for jj in range(4):
                    ce = plsc.load_gather(trow, [rowi, cols[jj]])
                    hbuf[i, pl.ds(jj * 32, 16)] = hbuf[i, pl.ds(jj * 32, 16)] * ce
                    hbuf[i, pl.ds(jj * 32 + 16, 16)] = (
                        hbuf[i, pl.ds(jj * 32 + 16, 16)] * ce)

            pltpu.sync_copy(hbuf, acc_sh.at[dst_v.at[q]], add=True)
        return carry

    lax.fori_loop(0, K2, chunk, 0)
    plsc.subcore_barrier()
    pltpu.sync_copy(acc_sh.at[pl.ds(s * ROWS_PT, ROWS_PT), :],
                    out_hbm.at[c, pl.ds(s * ROWS_PT, ROWS_PT), :])


def kernel(x, edge_index, batch, W1, as1, ad1, b1, W2, as2, ad2, b2,
           W3, as3, ad3, b3, fcW, fcb):
    f32 = jnp.float32
    loops = jnp.arange(N, dtype=jnp.int32)
    src = jnp.concatenate([edge_index[0].astype(jnp.int32), loops,
                           jnp.zeros((EP - EFULL,), jnp.int32)]).reshape(EP // 128, 128)
    dst = jnp.concatenate([edge_index[1].astype(jnp.int32), loops,
                           jnp.full((EP - EFULL,), DUMMY, jnp.int32)]).reshape(EP // 128, 128)
    xp = jnp.pad(x, ((0, NP - N), (0, 0)))
    zden = jnp.zeros((NP, 16), f32)
    zacc = jnp.zeros((NP, HALF), f32)
    eye8 = jnp.eye(NH, dtype=f32)

    def amat(a):
        m = (a[:, :, None] * eye8[:, None, :]).reshape(F, NH)
        return jnp.concatenate([m, m], axis=1)

    batchp = jnp.concatenate([batch.astype(jnp.int32),
                              jnp.full((NP - N,), G, jnp.int32)])
    batch3d = batchp.reshape(_GRID, 1, _R)

    hlo, hhi, As, Ad = _tc_first(xp, W1, amat(as1), amat(ad1))
    for (W, a_s, a_d, b_) in ((W2, as2, ad2, b1), (W3, as3, ad3, b2)):
        t_buf, den = _sc_pass1(src, dst, As, Ad, zden)
        out = _sc_pass2(src, dst, t_buf, _tc_denr(den[0], den[1]), hlo, hhi, zacc)
        hlo, hhi, As, Ad = _tc_mid(out[0], out[1], b_.reshape(1, F), W,
                                   amat(a_s), amat(a_d))
    t_buf, den = _sc_pass1(src, dst, As, Ad, zden)
    out = _sc_pass2(src, dst, t_buf, _tc_denr(den[0], den[1]), hlo, hhi, zacc)
    return _tc_pool(out[0], out[1], b3.reshape(1, F), batch3d, fcW,
                    fcb.reshape(1, NCLS))


# speedup vs baseline: 2.0734x; 1.8298x over previous
import functools
import jax
import jax.numpy as jnp
from jax import lax
from jax.experimental import pallas as pl
from jax.experimental.pallas import tpu as pltpu
from jax.experimental.pallas import tpu_sc as plsc

N = 10000
NP = 10240
F = 256
HALF = 128
NH = 8
G = 64
NCLS = 40
EFULL = 170000
T = 512
T1 = 256
K1 = 21
EP = 32 * T1 * K1
K2 = EP // (16 * T)
ROWS_PT = NP // 16
DUMMY = N

_R = 512
_GRID = NP // _R


def _tc_first_body(x_ref, w_ref, ms_ref, md_ref, hlo_ref, hhi_ref, as_ref, ad_ref):
    h = jnp.dot(x_ref[...], w_ref[...], preferred_element_type=jnp.float32)
    hlo_ref[...] = h[:, :HALF]
    hhi_ref[...] = h[:, HALF:]
    as_ref[...] = jnp.dot(h, ms_ref[...], preferred_element_type=jnp.float32)
    ad_ref[...] = jnp.dot(h, md_ref[...], preferred_element_type=jnp.float32)


def _tc_mid_body(plo_ref, phi_ref, b_ref, w_ref, ms_ref, md_ref,
                 hlo_ref, hhi_ref, as_ref, ad_ref):
    b = b_ref[...]
    alo = plo_ref[...] + b[:, :HALF]
    ahi = phi_ref[...] + b[:, HALF:]
    alo = jnp.where(alo > 0, alo, jnp.exp(alo) - 1.0)
    ahi = jnp.where(ahi > 0, ahi, jnp.exp(ahi) - 1.0)
    h = (jnp.dot(alo, w_ref[:HALF, :], preferred_element_type=jnp.float32)
         + jnp.dot(ahi, w_ref[HALF:, :], preferred_element_type=jnp.float32))
    hlo_ref[...] = h[:, :HALF]
    hhi_ref[...] = h[:, HALF:]
    as_ref[...] = jnp.dot(h, ms_ref[...], preferred_element_type=jnp.float32)
    ad_ref[...] = jnp.dot(h, md_ref[...], preferred_element_type=jnp.float32)


_TC_OUT_SHAPE = [jax.ShapeDtypeStruct((NP, HALF), jnp.float32),
                 jax.ShapeDtypeStruct((NP, HALF), jnp.float32),
                 jax.ShapeDtypeStruct((NP, 16), jnp.float32),
                 jax.ShapeDtypeStruct((NP, 16), jnp.float32)]
_TC_OUT_SPECS = [pl.BlockSpec((_R, HALF), lambda i: (i, 0)),
                 pl.BlockSpec((_R, HALF), lambda i: (i, 0)),
                 pl.BlockSpec((_R, 16), lambda i: (i, 0)),
                 pl.BlockSpec((_R, 16), lambda i: (i, 0))]


def _tc_first(xp, W, Ms, Md):
    return pl.pallas_call(
        _tc_first_body,
        grid=(_GRID,),
        in_specs=[pl.BlockSpec((_R, F), lambda i: (i, 0)),
                  pl.BlockSpec((F, F), lambda i: (0, 0)),
                  pl.BlockSpec((F, 16), lambda i: (0, 0)),
                  pl.BlockSpec((F, 16), lambda i: (0, 0))],
        out_specs=_TC_OUT_SPECS,
        out_shape=_TC_OUT_SHAPE,
    )(xp, W, Ms, Md)


def _tc_mid(plo, phi, b, W, Ms, Md):
    return pl.pallas_call(
        _tc_mid_body,
        grid=(_GRID,),
        in_specs=[pl.BlockSpec((_R, HALF), lambda i: (i, 0)),
                  pl.BlockSpec((_R, HALF), lambda i: (i, 0)),
                  pl.BlockSpec((1, F), lambda i: (0, 0)),
                  pl.BlockSpec((F, F), lambda i: (0, 0)),
                  pl.BlockSpec((F, 16), lambda i: (0, 0)),
                  pl.BlockSpec((F, 16), lambda i: (0, 0))],
        out_specs=_TC_OUT_SPECS,
        out_shape=_TC_OUT_SHAPE,
    )(plo, phi, b, W, Ms, Md)


def _pool_body(plo_ref, phi_ref, b_ref, batch_ref, fcw_ref, fcb_ref, out_ref,
               sum_ref, cnt_ref):
    i = pl.program_id(0)

    @pl.when(i == 0)
    def _():
        sum_ref[...] = jnp.zeros_like(sum_ref)
        cnt_ref[...] = jnp.zeros_like(cnt_ref)

    b = b_ref[...]
    y = jnp.concatenate([plo_ref[...] + b[:, :HALF], phi_ref[...] + b[:, HALF:]],
                        axis=1)
    bb = batch_ref[0]
    gi = lax.broadcasted_iota(jnp.int32, (G, _R), 0)
    oh = (gi == bb).astype(jnp.float32)
    sum_ref[...] += jnp.dot(oh, y, preferred_element_type=jnp.float32)
    cnt_ref[...] += jnp.broadcast_to(jnp.sum(oh, axis=1, keepdims=True), (G, HALF))

    @pl.when(i == _GRID - 1)
    def _():
        cnt = jnp.maximum(cnt_ref[...], 1.0)
        pooled = sum_ref[...] / jnp.concatenate([cnt, cnt], axis=1)
        out_ref[...] = (jnp.dot(pooled, fcw_ref[...],
                                preferred_element_type=jnp.float32) + fcb_ref[...])


def _tc_pool(plo, phi, b, batch3d, fcW, fcb):
    return pl.pallas_call(
        _pool_body,
        grid=(_GRID,),
        in_specs=[pl.BlockSpec((_R, HALF), lambda i: (i, 0)),
                  pl.BlockSpec((_R, HALF), lambda i: (i, 0)),
                  pl.BlockSpec((1, F), lambda i: (0, 0)),
                  pl.BlockSpec((1, 1, _R), lambda i: (i, 0, 0)),
                  pl.BlockSpec((F, NCLS), lambda i: (0, 0)),
                  pl.BlockSpec((1, NCLS), lambda i: (0, 0))],
        out_specs=pl.BlockSpec((G, NCLS), lambda i: (0, 0)),
        out_shape=jax.ShapeDtypeStruct((G, NCLS), jnp.float32),
        scratch_shapes=[pltpu.VMEM((G, F), jnp.float32),
                        pltpu.VMEM((G, HALF), jnp.float32)],
    )(plo, phi, b, batch3d, fcW, fcb)


_MESH = plsc.VectorSubcoreMesh(core_axis_name="c", subcore_axis_name="s")


def _denr_body(d0_ref, d1_ref, out_ref):
    out_ref[...] = 1.0 / (d0_ref[...] + d1_ref[...] + 1e-16)


def _tc_denr(d0, d1):
    return pl.pallas_call(
        _denr_body,
        grid=(_GRID,),
        in_specs=[pl.BlockSpec((_R, 16), lambda i: (i, 0)),
                  pl.BlockSpec((_R, 16), lambda i: (i, 0))],
        out_specs=pl.BlockSpec((_R, 16), lambda i: (i, 0)),
        out_shape=jax.ShapeDtypeStruct((NP, 16), jnp.float32),
    )(d0, d1)


@functools.partial(
    pl.kernel,
    out_type=[jax.ShapeDtypeStruct((EP, 16), jnp.float32),
              jax.ShapeDtypeStruct((2, NP, 16), jnp.float32)],
    mesh=_MESH,
    scratch_types=[pltpu.VMEM((2, 128), jnp.int32),
                   pltpu.VMEM((2, 128), jnp.int32),
                   pltpu.VMEM((T1, 16), jnp.float32),
                   pltpu.VMEM((T1, 16), jnp.float32),
                   pltpu.VMEM_SHARED((NP, 16), jnp.float32)],
    compiler_params=pltpu.CompilerParams(use_tc_tiling_on_sc=False, needs_layout_passes=False),
)
def _sc_pass1(src_hbm, dst_hbm, as_hbm, ad_hbm, zden_hbm, t_hbm, den_hbm,
              src_v, dst_v, srow, drow, den_sh):
    c = lax.axis_index("c")
    s = lax.axis_index("s")
    wid = s * 2 + c
    pltpu.sync_copy(zden_hbm.at[pl.ds(s * ROWS_PT, ROWS_PT), :],
                    den_sh.at[pl.ds(s * ROWS_PT, ROWS_PT), :])
    plsc.subcore_barrier()
    wbase = wid * (T1 * K1)

    def chunk(k, carry):
        base = wbase + k * T1
        brow = base // 128
        pltpu.sync_copy(src_hbm.at[pl.ds(brow, T1 // 128), :], src_v)
        pltpu.sync_copy(dst_hbm.at[pl.ds(brow, T1 // 128), :], dst_v)
        for q in range(T1 // 128):
            pltpu.sync_copy(as_hbm.at[src_v.at[q]],
                            srow.at[pl.ds(q * 128, 128), :])
            pltpu.sync_copy(ad_hbm.at[dst_v.at[q]],
                            drow.at[pl.ds(q * 128, 128), :])

        @plsc.parallel_loop(0, T1)
        def _row(i):
            v = srow[i, :] + drow[i, :]
            srow[i, :] = jnp.exp(jnp.maximum(v, 0.2 * v))

        pltpu.sync_copy(srow, t_hbm.at[pl.ds(base, T1), :])
        for q in range(T1 // 128):
            pltpu.sync_copy(srow.at[pl.ds(q * 128, 128), :],
                            den_sh.at[dst_v.at[q]], add=True)
        return carry

    lax.fori_loop(0, K1, chunk, 0)
    plsc.subcore_barrier()
    pltpu.sync_copy(den_sh.at[pl.ds(s * ROWS_PT, ROWS_PT), :],
                    den_hbm.at[c, pl.ds(s * ROWS_PT, ROWS_PT), :])


@functools.partial(
    pl.kernel,
    out_type=jax.ShapeDtypeStruct((2, NP, HALF), jnp.float32),
    mesh=_MESH,
    scratch_types=[pltpu.VMEM((4, T // 4), jnp.int32),
                   pltpu.VMEM((4, T // 4), jnp.int32),
                   pltpu.VMEM((T, 16), jnp.float32),
                   pltpu.VMEM((T, 16), jnp.float32),
                   pltpu.VMEM((128, HALF), jnp.float32),
                   pltpu.VMEM_SHARED((NP, HALF), jnp.float32)],
    compiler_params=pltpu.CompilerParams(use_tc_tiling_on_sc=False, needs_layout_passes=False),
)
def _sc_pass2(src_hbm, dst_hbm, t_hbm, denr_hbm, hlo_hbm, hhi_hbm,
              zacc_hbm, out_hbm,
              src_v, dst_v, trow, dr, hbuf, acc_sh):
    c = lax.axis_index("c")
    s = lax.axis_index("s")
    pltpu.sync_copy(zacc_hbm.at[pl.ds(s * ROWS_PT, ROWS_PT), :],
                    acc_sh.at[pl.ds(s * ROWS_PT, ROWS_PT), :])
    plsc.subcore_barrier()
    tbase = s * (T * K2)
    hb = c * 4
    cols = [jnp.full((16,), hb + jj, jnp.int32) for jj in range(4)]

    def chunk(k, carry):
        base = tbase + k * T
        brow = base // 128
        pltpu.sync_copy(src_hbm.at[pl.ds(brow, T // 128), :], src_v)
        pltpu.sync_copy(dst_hbm.at[pl.ds(brow, T // 128), :], dst_v)
        pltpu.sync_copy(t_hbm.at[pl.ds(base, T), :], trow)
        for q in range(T // 128):
            pltpu.sync_copy(denr_hbm.at[dst_v.at[q]],
                            dr.at[pl.ds(q * 128, 128), :])

        @plsc.parallel_loop(0, T)
        def _coef_row(i):
            trow[i, :] = trow[i, :] * dr[i, :]

        for q in range(T // 128):
            @pl.when(c == 0)
            def _():
                pltpu.sync_copy(hlo_hbm.at[src_v.at[q]], hbuf)

            @pl.when(c != 0)
            def _():
                pltpu.sync_copy(hhi_hbm.at[src_v.at[q]], hbuf)

            @plsc.parallel_loop(0, 128)
            def _scale_row(i):
                rowi = jnp.full((16,), q * 128 + i, jnp.int32)
                for jj in range(4):
                    ce = plsc.load_gather(trow, [rowi, cols[jj]])
                    hbuf[i, pl.ds(jj * 32, 16)] = hbuf[i, pl.ds(jj * 32, 16)] * ce
                    hbuf[i, pl.ds(jj * 32 + 16, 16)] = (
                        hbuf[i, pl.ds(jj * 32 + 16, 16)] * ce)

            pltpu.sync_copy(hbuf, acc_sh.at[dst_v.at[q]], add=True)
        return carry

    lax.fori_loop(0, K2, chunk, 0)
    plsc.subcore_barrier()
    pltpu.sync_copy(acc_sh.at[pl.ds(s * ROWS_PT, ROWS_PT), :],
                    out_hbm.at[c, pl.ds(s * ROWS_PT, ROWS_PT), :])


def kernel(x, edge_index, batch, W1, as1, ad1, b1, W2, as2, ad2, b2,
           W3, as3, ad3, b3, fcW, fcb):
    f32 = jnp.float32
    loops = jnp.arange(N, dtype=jnp.int32)
    src = jnp.concatenate([edge_index[0].astype(jnp.int32), loops,
                           jnp.zeros((EP - EFULL,), jnp.int32)]).reshape(EP // 128, 128)
    dst = jnp.concatenate([edge_index[1].astype(jnp.int32), loops,
                           jnp.full((EP - EFULL,), DUMMY, jnp.int32)]).reshape(EP // 128, 128)
    xp = jnp.pad(x, ((0, NP - N), (0, 0)))
    zden = jnp.zeros((NP, 16), f32)
    zacc = jnp.zeros((NP, HALF), f32)
    eye8 = jnp.eye(NH, dtype=f32)

    def amat(a):
        m = (a[:, :, None] * eye8[:, None, :]).reshape(F, NH)
        return jnp.concatenate([m, m], axis=1)

    batchp = jnp.concatenate([batch.astype(jnp.int32),
                              jnp.full((NP - N,), G, jnp.int32)])
    batch3d = batchp.reshape(_GRID, 1, _R)

    hlo, hhi, As, Ad = _tc_first(xp, W1, amat(as1), amat(ad1))
    for (W, a_s, a_d, b_) in ((W2, as2, ad2, b1), (W3, as3, ad3, b2)):
        t_buf, den = _sc_pass1(src, dst, As, Ad, zden)
        out = _sc_pass2(src, dst, t_buf, _tc_denr(den[0], den[1]), hlo, hhi, zacc)
        hlo, hhi, As, Ad = _tc_mid(out[0], out[1], b_.reshape(1, F), W,
                                   amat(a_s), amat(a_d))
    t_buf, den = _sc_pass1(src, dst, As, Ad, zden)
    out = _sc_pass2(src, dst, t_buf, _tc_denr(den[0], den[1]), hlo, hhi, zacc)
    return _tc_pool(out[0], out[1], b3.reshape(1, F), batch3d, fcW,
                    fcb.reshape(1, NCLS))


# speedup vs baseline: 2.3936x; 1.1544x over previous
import functools
import jax
import jax.numpy as jnp
from jax import lax
from jax.experimental import pallas as pl
from jax.experimental.pallas import tpu as pltpu
from jax.experimental.pallas import tpu_sc as plsc

N = 10000
NP = 10240
F = 256
HALF = 128
NH = 8
G = 64
NCLS = 40
EFULL = 170000
T = 512
T1 = 256
K1 = 21
EP = 32 * T1 * K1
K2 = EP // (16 * T)
ROWS_PT = NP // 16
DUMMY = N

_R = 512
_GRID = NP // _R


def _tc_first_body(x_ref, w_ref, ms_ref, md_ref, hlo_ref, hhi_ref, as_ref, ad_ref):
    h = jnp.dot(x_ref[...], w_ref[...], preferred_element_type=jnp.float32)
    hlo_ref[...] = h[:, :HALF]
    hhi_ref[...] = h[:, HALF:]
    as_ref[...] = jnp.dot(h, ms_ref[...], preferred_element_type=jnp.float32)
    ad_ref[...] = jnp.dot(h, md_ref[...], preferred_element_type=jnp.float32)


def _tc_mid_body(plo_ref, phi_ref, b_ref, w_ref, ms_ref, md_ref,
                 hlo_ref, hhi_ref, as_ref, ad_ref):
    b = b_ref[...]
    alo = plo_ref[...] + b[:, :HALF]
    ahi = phi_ref[...] + b[:, HALF:]
    alo = jnp.where(alo > 0, alo, jnp.exp(alo) - 1.0)
    ahi = jnp.where(ahi > 0, ahi, jnp.exp(ahi) - 1.0)
    h = (jnp.dot(alo, w_ref[:HALF, :], preferred_element_type=jnp.float32)
         + jnp.dot(ahi, w_ref[HALF:, :], preferred_element_type=jnp.float32))
    hlo_ref[...] = h[:, :HALF]
    hhi_ref[...] = h[:, HALF:]
    as_ref[...] = jnp.dot(h, ms_ref[...], preferred_element_type=jnp.float32)
    ad_ref[...] = jnp.dot(h, md_ref[...], preferred_element_type=jnp.float32)


_TC_OUT_SHAPE = [jax.ShapeDtypeStruct((NP, HALF), jnp.float32),
                 jax.ShapeDtypeStruct((NP, HALF), jnp.float32),
                 jax.ShapeDtypeStruct((NP, 16), jnp.float32),
                 jax.ShapeDtypeStruct((NP, 16), jnp.float32)]
_TC_OUT_SPECS = [pl.BlockSpec((_R, HALF), lambda i: (i, 0)),
                 pl.BlockSpec((_R, HALF), lambda i: (i, 0)),
                 pl.BlockSpec((_R, 16), lambda i: (i, 0)),
                 pl.BlockSpec((_R, 16), lambda i: (i, 0))]


def _tc_first(xp, W, Ms, Md):
    return pl.pallas_call(
        _tc_first_body,
        grid=(_GRID,),
        in_specs=[pl.BlockSpec((_R, F), lambda i: (i, 0)),
                  pl.BlockSpec((F, F), lambda i: (0, 0)),
                  pl.BlockSpec((F, 16), lambda i: (0, 0)),
                  pl.BlockSpec((F, 16), lambda i: (0, 0))],
        out_specs=_TC_OUT_SPECS,
        out_shape=_TC_OUT_SHAPE,
    )(xp, W, Ms, Md)


def _tc_mid(plo, phi, b, W, Ms, Md):
    return pl.pallas_call(
        _tc_mid_body,
        grid=(_GRID,),
        in_specs=[pl.BlockSpec((_R, HALF), lambda i: (i, 0)),
                  pl.BlockSpec((_R, HALF), lambda i: (i, 0)),
                  pl.BlockSpec((1, F), lambda i: (0, 0)),
                  pl.BlockSpec((F, F), lambda i: (0, 0)),
                  pl.BlockSpec((F, 16), lambda i: (0, 0)),
                  pl.BlockSpec((F, 16), lambda i: (0, 0))],
        out_specs=_TC_OUT_SPECS,
        out_shape=_TC_OUT_SHAPE,
    )(plo, phi, b, W, Ms, Md)


def _pool_body(plo_ref, phi_ref, b_ref, batch_ref, fcw_ref, fcb_ref, out_ref,
               sum_ref, cnt_ref):
    i = pl.program_id(0)

    @pl.when(i == 0)
    def _():
        sum_ref[...] = jnp.zeros_like(sum_ref)
        cnt_ref[...] = jnp.zeros_like(cnt_ref)

    b = b_ref[...]
    y = jnp.concatenate([plo_ref[...] + b[:, :HALF], phi_ref[...] + b[:, HALF:]],
                        axis=1)
    bb = batch_ref[0]
    gi = lax.broadcasted_iota(jnp.int32, (G, _R), 0)
    oh = (gi == bb).astype(jnp.float32)
    sum_ref[...] += jnp.dot(oh, y, preferred_element_type=jnp.float32)
    cnt_ref[...] += jnp.broadcast_to(jnp.sum(oh, axis=1, keepdims=True), (G, HALF))

    @pl.when(i == _GRID - 1)
    def _():
        cnt = jnp.maximum(cnt_ref[...], 1.0)
        pooled = sum_ref[...] / jnp.concatenate([cnt, cnt], axis=1)
        out_ref[...] = (jnp.dot(pooled, fcw_ref[...],
                                preferred_element_type=jnp.float32) + fcb_ref[...])


def _tc_pool(plo, phi, b, batch3d, fcW, fcb):
    return pl.pallas_call(
        _pool_body,
        grid=(_GRID,),
        in_specs=[pl.BlockSpec((_R, HALF), lambda i: (i, 0)),
                  pl.BlockSpec((_R, HALF), lambda i: (i, 0)),
                  pl.BlockSpec((1, F), lambda i: (0, 0)),
                  pl.BlockSpec((1, 1, _R), lambda i: (i, 0, 0)),
                  pl.BlockSpec((F, NCLS), lambda i: (0, 0)),
                  pl.BlockSpec((1, NCLS), lambda i: (0, 0))],
        out_specs=pl.BlockSpec((G, NCLS), lambda i: (0, 0)),
        out_shape=jax.ShapeDtypeStruct((G, NCLS), jnp.float32),
        scratch_shapes=[pltpu.VMEM((G, F), jnp.float32),
                        pltpu.VMEM((G, HALF), jnp.float32)],
    )(plo, phi, b, batch3d, fcW, fcb)


_MESH = plsc.VectorSubcoreMesh(core_axis_name="c", subcore_axis_name="s")


def _denr_body(d0_ref, d1_ref, out_ref):
    out_ref[...] = 1.0 / (d0_ref[...] + d1_ref[...] + 1e-16)


def _tc_denr(d0, d1):
    return pl.pallas_call(
        _denr_body,
        grid=(_GRID,),
        in_specs=[pl.BlockSpec((_R, 16), lambda i: (i, 0)),
                  pl.BlockSpec((_R, 16), lambda i: (i, 0))],
        out_specs=pl.BlockSpec((_R, 16), lambda i: (i, 0)),
        out_shape=jax.ShapeDtypeStruct((NP, 16), jnp.float32),
    )(d0, d1)


@functools.partial(
    pl.kernel,
    out_type=[jax.ShapeDtypeStruct((EP, 16), jnp.float32),
              jax.ShapeDtypeStruct((2, NP, 16), jnp.float32)],
    mesh=_MESH,
    scratch_types=[pltpu.VMEM((2, 128), jnp.int32),
                   pltpu.VMEM((2, 128), jnp.int32),
                   pltpu.VMEM((T1, 16), jnp.float32),
                   pltpu.VMEM((T1, 16), jnp.float32),
                   pltpu.VMEM_SHARED((NP, 16), jnp.float32),
                   pltpu.VMEM_SHARED((NP, 16), jnp.float32),
                   pltpu.VMEM_SHARED((NP, 16), jnp.float32)],
    compiler_params=pltpu.CompilerParams(use_tc_tiling_on_sc=False, needs_layout_passes=False),
)
def _sc_pass1(src_hbm, dst_hbm, as_hbm, ad_hbm, zden_hbm, t_hbm, den_hbm,
              src_v, dst_v, srow, drow, den_sh, as_sh, ad_sh):
    c = lax.axis_index("c")
    s = lax.axis_index("s")
    wid = s * 2 + c
    rs = pl.ds(s * ROWS_PT, ROWS_PT)
    pltpu.sync_copy(zden_hbm.at[rs, :], den_sh.at[rs, :])
    pltpu.sync_copy(as_hbm.at[rs, :], as_sh.at[rs, :])
    pltpu.sync_copy(ad_hbm.at[rs, :], ad_sh.at[rs, :])
    plsc.subcore_barrier()
    wbase = wid * (T1 * K1)

    def chunk(k, carry):
        base = wbase + k * T1
        brow = base // 128
        pltpu.sync_copy(src_hbm.at[pl.ds(brow, T1 // 128), :], src_v)
        pltpu.sync_copy(dst_hbm.at[pl.ds(brow, T1 // 128), :], dst_v)
        for q in range(T1 // 128):
            pltpu.sync_copy(as_sh.at[src_v.at[q]],
                            srow.at[pl.ds(q * 128, 128), :])
            pltpu.sync_copy(ad_sh.at[dst_v.at[q]],
                            drow.at[pl.ds(q * 128, 128), :])

        @plsc.parallel_loop(0, T1)
        def _row(i):
            v = srow[i, :] + drow[i, :]
            srow[i, :] = jnp.exp(jnp.maximum(v, 0.2 * v))

        pltpu.sync_copy(srow, t_hbm.at[pl.ds(base, T1), :])
        for q in range(T1 // 128):
            pltpu.sync_copy(srow.at[pl.ds(q * 128, 128), :],
                            den_sh.at[dst_v.at[q]], add=True)
        return carry

    lax.fori_loop(0, K1, chunk, 0)
    plsc.subcore_barrier()
    pltpu.sync_copy(den_sh.at[rs, :], den_hbm.at[c, rs, :])


@functools.partial(
    pl.kernel,
    out_type=jax.ShapeDtypeStruct((2, NP, HALF), jnp.float32),
    mesh=_MESH,
    scratch_types=[pltpu.VMEM((4, T // 4), jnp.int32),
                   pltpu.VMEM((4, T // 4), jnp.int32),
                   pltpu.VMEM((T, 16), jnp.float32),
                   pltpu.VMEM((T, 16), jnp.float32),
                   pltpu.VMEM((128, HALF), jnp.float32),
                   pltpu.VMEM_SHARED((NP, HALF), jnp.float32),
                   pltpu.VMEM_SHARED((NP, 16), jnp.float32)],
    compiler_params=pltpu.CompilerParams(use_tc_tiling_on_sc=False, needs_layout_passes=False),
)
def _sc_pass2(src_hbm, dst_hbm, t_hbm, denr_hbm, hlo_hbm, hhi_hbm,
              zacc_hbm, out_hbm,
              src_v, dst_v, trow, dr, hbuf, acc_sh, denr_sh):
    c = lax.axis_index("c")
    s = lax.axis_index("s")
    rs = pl.ds(s * ROWS_PT, ROWS_PT)
    pltpu.sync_copy(zacc_hbm.at[rs, :], acc_sh.at[rs, :])
    pltpu.sync_copy(denr_hbm.at[rs, :], denr_sh.at[rs, :])
    plsc.subcore_barrier()
    tbase = s * (T * K2)
    hb = c * 4
    cols = [jnp.full((16,), hb + jj, jnp.int32) for jj in range(4)]

    def chunk(k, carry):
        base = tbase + k * T
        brow = base // 128
        pltpu.sync_copy(src_hbm.at[pl.ds(brow, T // 128), :], src_v)
        pltpu.sync_copy(dst_hbm.at[pl.ds(brow, T // 128), :], dst_v)
        pltpu.sync_copy(t_hbm.at[pl.ds(base, T), :], trow)
        for q in range(T // 128):
            pltpu.sync_copy(denr_sh.at[dst_v.at[q]],
                            dr.at[pl.ds(q * 128, 128), :])

        @plsc.parallel_loop(0, T)
        def _coef_row(i):
            trow[i, :] = trow[i, :] * dr[i, :]

        for q in range(T // 128):
            @pl.when(c == 0)
            def _():
                pltpu.sync_copy(hlo_hbm.at[src_v.at[q]], hbuf)

            @pl.when(c != 0)
            def _():
                pltpu.sync_copy(hhi_hbm.at[src_v.at[q]], hbuf)

            @plsc.parallel_loop(0, 128)
            def _scale_row(i):
                rowi = jnp.full((16,), q * 128 + i, jnp.int32)
                for jj in range(4):
                    ce = plsc.load_gather(trow, [rowi, cols[jj]])
                    hbuf[i, pl.ds(jj * 32, 16)] = hbuf[i, pl.ds(jj * 32, 16)] * ce
                    hbuf[i, pl.ds(jj * 32 + 16, 16)] = (
                        hbuf[i, pl.ds(jj * 32 + 16, 16)] * ce)

            pltpu.sync_copy(hbuf, acc_sh.at[dst_v.at[q]], add=True)
        return carry

    lax.fori_loop(0, K2, chunk, 0)
    plsc.subcore_barrier()
    pltpu.sync_copy(acc_sh.at[rs, :], out_hbm.at[c, rs, :])


def kernel(x, edge_index, batch, W1, as1, ad1, b1, W2, as2, ad2, b2,
           W3, as3, ad3, b3, fcW, fcb):
    f32 = jnp.float32
    loops = jnp.arange(N, dtype=jnp.int32)
    src = jnp.concatenate([edge_index[0].astype(jnp.int32), loops,
                           jnp.zeros((EP - EFULL,), jnp.int32)]).reshape(EP // 128, 128)
    dst = jnp.concatenate([edge_index[1].astype(jnp.int32), loops,
                           jnp.full((EP - EFULL,), DUMMY, jnp.int32)]).reshape(EP // 128, 128)
    xp = jnp.pad(x, ((0, NP - N), (0, 0)))
    zden = jnp.zeros((NP, 16), f32)
    zacc = jnp.zeros((NP, HALF), f32)
    eye8 = jnp.eye(NH, dtype=f32)

    def amat(a):
        m = (a[:, :, None] * eye8[:, None, :]).reshape(F, NH)
        return jnp.concatenate([m, m], axis=1)

    batchp = jnp.concatenate([batch.astype(jnp.int32),
                              jnp.full((NP - N,), G, jnp.int32)])
    batch3d = batchp.reshape(_GRID, 1, _R)

    hlo, hhi, As, Ad = _tc_first(xp, W1, amat(as1), amat(ad1))
    for (W, a_s, a_d, b_) in ((W2, as2, ad2, b1), (W3, as3, ad3, b2)):
        t_buf, den = _sc_pass1(src, dst, As, Ad, zden)
        out = _sc_pass2(src, dst, t_buf, _tc_denr(den[0], den[1]), hlo, hhi, zacc)
        hlo, hhi, As, Ad = _tc_mid(out[0], out[1], b_.reshape(1, F), W,
                                   amat(a_s), amat(a_d))
    t_buf, den = _sc_pass1(src, dst, As, Ad, zden)
    out = _sc_pass2(src, dst, t_buf, _tc_denr(den[0], den[1]), hlo, hhi, zacc)
    return _tc_pool(out[0], out[1], b3.reshape(1, F), batch3d, fcW,
                    fcb.reshape(1, NCLS))


# speedup vs baseline: 2.6404x; 1.1031x over previous
import functools
import jax
import jax.numpy as jnp
from jax import lax
from jax.experimental import pallas as pl
from jax.experimental.pallas import tpu as pltpu
from jax.experimental.pallas import tpu_sc as plsc

N = 10000
NP = 10240
F = 256
HALF = 128
NH = 8
G = 64
NCLS = 40
EFULL = 170000
T = 512
T1 = 256
K1 = 21
EP = 32 * T1 * K1
T2 = 256
K2 = EP // (16 * T2)
ROWS_PT = NP // 16
DUMMY = N

_R = 512
_GRID = NP // _R


def _tc_first_body(x_ref, w_ref, ms_ref, md_ref, hlo_ref, hhi_ref, as_ref, ad_ref):
    h = jnp.dot(x_ref[...], w_ref[...], preferred_element_type=jnp.float32)
    hlo_ref[...] = h[:, :HALF]
    hhi_ref[...] = h[:, HALF:]
    as_ref[...] = jnp.dot(h, ms_ref[...], preferred_element_type=jnp.float32)
    ad_ref[...] = jnp.dot(h, md_ref[...], preferred_element_type=jnp.float32)


def _tc_mid_body(plo_ref, phi_ref, b_ref, w_ref, ms_ref, md_ref,
                 hlo_ref, hhi_ref, as_ref, ad_ref):
    b = b_ref[...]
    alo = plo_ref[...] + b[:, :HALF]
    ahi = phi_ref[...] + b[:, HALF:]
    alo = jnp.where(alo > 0, alo, jnp.exp(alo) - 1.0)
    ahi = jnp.where(ahi > 0, ahi, jnp.exp(ahi) - 1.0)
    h = (jnp.dot(alo, w_ref[:HALF, :], preferred_element_type=jnp.float32)
         + jnp.dot(ahi, w_ref[HALF:, :], preferred_element_type=jnp.float32))
    hlo_ref[...] = h[:, :HALF]
    hhi_ref[...] = h[:, HALF:]
    as_ref[...] = jnp.dot(h, ms_ref[...], preferred_element_type=jnp.float32)
    ad_ref[...] = jnp.dot(h, md_ref[...], preferred_element_type=jnp.float32)


_TC_OUT_SHAPE = [jax.ShapeDtypeStruct((NP, HALF), jnp.float32),
                 jax.ShapeDtypeStruct((NP, HALF), jnp.float32),
                 jax.ShapeDtypeStruct((NP, 16), jnp.float32),
                 jax.ShapeDtypeStruct((NP, 16), jnp.float32)]
_TC_OUT_SPECS = [pl.BlockSpec((_R, HALF), lambda i: (i, 0)),
                 pl.BlockSpec((_R, HALF), lambda i: (i, 0)),
                 pl.BlockSpec((_R, 16), lambda i: (i, 0)),
                 pl.BlockSpec((_R, 16), lambda i: (i, 0))]


def _tc_first(xp, W, Ms, Md):
    return pl.pallas_call(
        _tc_first_body,
        grid=(_GRID,),
        in_specs=[pl.BlockSpec((_R, F), lambda i: (i, 0)),
                  pl.BlockSpec((F, F), lambda i: (0, 0)),
                  pl.BlockSpec((F, 16), lambda i: (0, 0)),
                  pl.BlockSpec((F, 16), lambda i: (0, 0))],
        out_specs=_TC_OUT_SPECS,
        out_shape=_TC_OUT_SHAPE,
    )(xp, W, Ms, Md)


def _tc_mid(plo, phi, b, W, Ms, Md):
    return pl.pallas_call(
        _tc_mid_body,
        grid=(_GRID,),
        in_specs=[pl.BlockSpec((_R, HALF), lambda i: (i, 0)),
                  pl.BlockSpec((_R, HALF), lambda i: (i, 0)),
                  pl.BlockSpec((1, F), lambda i: (0, 0)),
                  pl.BlockSpec((F, F), lambda i: (0, 0)),
                  pl.BlockSpec((F, 16), lambda i: (0, 0)),
                  pl.BlockSpec((F, 16), lambda i: (0, 0))],
        out_specs=_TC_OUT_SPECS,
        out_shape=_TC_OUT_SHAPE,
    )(plo, phi, b, W, Ms, Md)


def _pool_body(plo_ref, phi_ref, b_ref, batch_ref, fcw_ref, fcb_ref, out_ref,
               sum_ref, cnt_ref):
    i = pl.program_id(0)

    @pl.when(i == 0)
    def _():
        sum_ref[...] = jnp.zeros_like(sum_ref)
        cnt_ref[...] = jnp.zeros_like(cnt_ref)

    b = b_ref[...]
    y = jnp.concatenate([plo_ref[...] + b[:, :HALF], phi_ref[...] + b[:, HALF:]],
                        axis=1)
    bb = batch_ref[0]
    gi = lax.broadcasted_iota(jnp.int32, (G, _R), 0)
    oh = (gi == bb).astype(jnp.float32)
    sum_ref[...] += jnp.dot(oh, y, preferred_element_type=jnp.float32)
    cnt_ref[...] += jnp.broadcast_to(jnp.sum(oh, axis=1, keepdims=True), (G, HALF))

    @pl.when(i == _GRID - 1)
    def _():
        cnt = jnp.maximum(cnt_ref[...], 1.0)
        pooled = sum_ref[...] / jnp.concatenate([cnt, cnt], axis=1)
        out_ref[...] = (jnp.dot(pooled, fcw_ref[...],
                                preferred_element_type=jnp.float32) + fcb_ref[...])


def _tc_pool(plo, phi, b, batch3d, fcW, fcb):
    return pl.pallas_call(
        _pool_body,
        grid=(_GRID,),
        in_specs=[pl.BlockSpec((_R, HALF), lambda i: (i, 0)),
                  pl.BlockSpec((_R, HALF), lambda i: (i, 0)),
                  pl.BlockSpec((1, F), lambda i: (0, 0)),
                  pl.BlockSpec((1, 1, _R), lambda i: (i, 0, 0)),
                  pl.BlockSpec((F, NCLS), lambda i: (0, 0)),
                  pl.BlockSpec((1, NCLS), lambda i: (0, 0))],
        out_specs=pl.BlockSpec((G, NCLS), lambda i: (0, 0)),
        out_shape=jax.ShapeDtypeStruct((G, NCLS), jnp.float32),
        scratch_shapes=[pltpu.VMEM((G, F), jnp.float32),
                        pltpu.VMEM((G, HALF), jnp.float32)],
    )(plo, phi, b, batch3d, fcW, fcb)


_MESH = plsc.VectorSubcoreMesh(core_axis_name="c", subcore_axis_name="s")


def _denr_body(d0_ref, d1_ref, out_ref):
    out_ref[...] = 1.0 / (d0_ref[...] + d1_ref[...] + 1e-16)


def _tc_denr(d0, d1):
    return pl.pallas_call(
        _denr_body,
        grid=(_GRID,),
        in_specs=[pl.BlockSpec((_R, 16), lambda i: (i, 0)),
                  pl.BlockSpec((_R, 16), lambda i: (i, 0))],
        out_specs=pl.BlockSpec((_R, 16), lambda i: (i, 0)),
        out_shape=jax.ShapeDtypeStruct((NP, 16), jnp.float32),
    )(d0, d1)


@functools.partial(
    pl.kernel,
    out_type=[jax.ShapeDtypeStruct((EP, 16), jnp.float32),
              jax.ShapeDtypeStruct((2, NP, 16), jnp.float32)],
    mesh=_MESH,
    scratch_types=[pltpu.VMEM((2, 128), jnp.int32),
                   pltpu.VMEM((2, 128), jnp.int32),
                   pltpu.VMEM((T1, 16), jnp.float32),
                   pltpu.VMEM((T1, 16), jnp.float32),
                   pltpu.VMEM_SHARED((NP, 16), jnp.float32),
                   pltpu.VMEM_SHARED((NP, 16), jnp.float32),
                   pltpu.VMEM_SHARED((NP, 16), jnp.float32)],
    compiler_params=pltpu.CompilerParams(use_tc_tiling_on_sc=False, needs_layout_passes=False),
)
def _sc_pass1(src_hbm, dst_hbm, as_hbm, ad_hbm, zden_hbm, t_hbm, den_hbm,
              src_v, dst_v, srow, drow, den_sh, as_sh, ad_sh):
    c = lax.axis_index("c")
    s = lax.axis_index("s")
    wid = s * 2 + c
    rs = pl.ds(s * ROWS_PT, ROWS_PT)
    pltpu.sync_copy(zden_hbm.at[rs, :], den_sh.at[rs, :])
    pltpu.sync_copy(as_hbm.at[rs, :], as_sh.at[rs, :])
    pltpu.sync_copy(ad_hbm.at[rs, :], ad_sh.at[rs, :])
    plsc.subcore_barrier()
    wbase = wid * (T1 * K1)

    def chunk(k, carry):
        base = wbase + k * T1
        brow = base // 128
        pltpu.sync_copy(src_hbm.at[pl.ds(brow, T1 // 128), :], src_v)
        pltpu.sync_copy(dst_hbm.at[pl.ds(brow, T1 // 128), :], dst_v)
        for q in range(T1 // 128):
            pltpu.sync_copy(as_sh.at[src_v.at[q]],
                            srow.at[pl.ds(q * 128, 128), :])
            pltpu.sync_copy(ad_sh.at[dst_v.at[q]],
                            drow.at[pl.ds(q * 128, 128), :])

        @plsc.parallel_loop(0, T1)
        def _row(i):
            v = srow[i, :] + drow[i, :]
            srow[i, :] = jnp.exp(jnp.maximum(v, 0.2 * v))

        pltpu.sync_copy(srow, t_hbm.at[pl.ds(base, T1), :])
        for q in range(T1 // 128):
            pltpu.sync_copy(srow.at[pl.ds(q * 128, 128), :],
                            den_sh.at[dst_v.at[q]], add=True)
        return carry

    lax.fori_loop(0, K1, chunk, 0)
    plsc.subcore_barrier()
    pltpu.sync_copy(den_sh.at[rs, :], den_hbm.at[c, rs, :])


@functools.partial(
    pl.kernel,
    out_type=jax.ShapeDtypeStruct((2, NP, HALF), jnp.float32),
    mesh=_MESH,
    scratch_types=[pltpu.VMEM((2, 128), jnp.int32),
                   pltpu.VMEM((2, 128), jnp.int32),
                   pltpu.VMEM((T2, 16), jnp.float32),
                   pltpu.VMEM((T2, 16), jnp.float32),
                   pltpu.VMEM((2, 128, HALF), jnp.float32),
                   pltpu.VMEM_SHARED((NP, HALF), jnp.float32),
                   pltpu.SemaphoreType.DMA,
                   pltpu.SemaphoreType.DMA],
    compiler_params=pltpu.CompilerParams(use_tc_tiling_on_sc=False, needs_layout_passes=False),
)
def _sc_pass2(src_hbm, dst_hbm, t_hbm, denr_hbm, hlo_hbm, hhi_hbm,
              zacc_hbm, out_hbm,
              src_v, dst_v, trow, dr, hbuf, acc_sh, sem_t, sem_h):
    c = lax.axis_index("c")
    s = lax.axis_index("s")
    rs = pl.ds(s * ROWS_PT, ROWS_PT)
    pltpu.sync_copy(zacc_hbm.at[rs, :], acc_sh.at[rs, :])
    plsc.subcore_barrier()
    tbase = s * (T2 * K2)
    hb = c * 4
    cols = [jnp.full((16,), hb + jj, jnp.int32) for jj in range(4)]

    def chunk(k, carry):
        base = tbase + k * T2
        brow = base // 128
        pltpu.sync_copy(src_hbm.at[pl.ds(brow, T2 // 128), :], src_v)
        pltpu.sync_copy(dst_hbm.at[pl.ds(brow, T2 // 128), :], dst_v)
        dt = pltpu.async_copy(t_hbm.at[pl.ds(base, T2), :], trow, sem_t)
        hs = []

        @pl.when(c == 0)
        def _():
            hs.append(pltpu.async_copy(hlo_hbm.at[src_v.at[0]],
                                       hbuf.at[0], sem_h))

        @pl.when(c != 0)
        def _():
            hs.append(pltpu.async_copy(hhi_hbm.at[src_v.at[0]],
                                       hbuf.at[0], sem_h))

        for q in range(T2 // 128):
            pltpu.sync_copy(denr_hbm.at[dst_v.at[q]],
                            dr.at[pl.ds(q * 128, 128), :])
        dt.wait()

        @plsc.parallel_loop(0, T2)
        def _coef_row(i):
            trow[i, :] = trow[i, :] * dr[i, :]

        for q in range(T2 // 128):
            hs[0].wait()
            hs.pop()
            if q + 1 < T2 // 128:
                @pl.when(c == 0)
                def _():
                    hs.append(pltpu.async_copy(hlo_hbm.at[src_v.at[q + 1]],
                                               hbuf.at[q + 1], sem_h))

                @pl.when(c != 0)
                def _():
                    hs.append(pltpu.async_copy(hhi_hbm.at[src_v.at[q + 1]],
                                               hbuf.at[q + 1], sem_h))

            @plsc.parallel_loop(0, 128)
            def _scale_row(i):
                rowi = jnp.full((16,), q * 128 + i, jnp.int32)
                for jj in range(4):
                    ce = plsc.load_gather(trow, [rowi, cols[jj]])
                    hbuf[q, i, pl.ds(jj * 32, 16)] = (
                        hbuf[q, i, pl.ds(jj * 32, 16)] * ce)
                    hbuf[q, i, pl.ds(jj * 32 + 16, 16)] = (
                        hbuf[q, i, pl.ds(jj * 32 + 16, 16)] * ce)

            pltpu.sync_copy(hbuf.at[q], acc_sh.at[dst_v.at[q]], add=True)
        return carry

    lax.fori_loop(0, K2, chunk, 0)
    plsc.subcore_barrier()
    pltpu.sync_copy(acc_sh.at[rs, :], out_hbm.at[c, rs, :])


def kernel(x, edge_index, batch, W1, as1, ad1, b1, W2, as2, ad2, b2,
           W3, as3, ad3, b3, fcW, fcb):
    f32 = jnp.float32
    loops = jnp.arange(N, dtype=jnp.int32)
    src = jnp.concatenate([edge_index[0].astype(jnp.int32), loops,
                           jnp.zeros((EP - EFULL,), jnp.int32)]).reshape(EP // 128, 128)
    dst = jnp.concatenate([edge_index[1].astype(jnp.int32), loops,
                           jnp.full((EP - EFULL,), DUMMY, jnp.int32)]).reshape(EP // 128, 128)
    xp = jnp.pad(x, ((0, NP - N), (0, 0)))
    zden = jnp.zeros((NP, 16), f32)
    zacc = jnp.zeros((NP, HALF), f32)
    eye8 = jnp.eye(NH, dtype=f32)

    def amat(a):
        m = (a[:, :, None] * eye8[:, None, :]).reshape(F, NH)
        return jnp.concatenate([m, m], axis=1)

    batchp = jnp.concatenate([batch.astype(jnp.int32),
                              jnp.full((NP - N,), G, jnp.int32)])
    batch3d = batchp.reshape(_GRID, 1, _R)

    hlo, hhi, As, Ad = _tc_first(xp, W1, amat(as1), amat(ad1))
    for (W, a_s, a_d, b_) in ((W2, as2, ad2, b1), (W3, as3, ad3, b2)):
        t_buf, den = _sc_pass1(src, dst, As, Ad, zden)
        out = _sc_pass2(src, dst, t_buf, _tc_denr(den[0], den[1]), hlo, hhi, zacc)
        hlo, hhi, As, Ad = _tc_mid(out[0], out[1], b_.reshape(1, F), W,
                                   amat(a_s), amat(a_d))
    t_buf, den = _sc_pass1(src, dst, As, Ad, zden)
    out = _sc_pass2(src, dst, t_buf, _tc_denr(den[0], den[1]), hlo, hhi, zacc)
    return _tc_pool(out[0], out[1], b3.reshape(1, F), batch3d, fcW,
                    fcb.reshape(1, NCLS))
